# baseline (speedup 1.0000x reference)
import jax
import jax.numpy as jnp
from jax import lax
from jax.experimental import pallas as pl
from jax.experimental.pallas import tpu as pltpu
from jax.experimental.pallas import tpu_sc as plsc

VOCAB = 1000000
EMBED_DIM = 64
BATCH = 16384
HIST = 20

_NC = 2
_NS = 16
_NW = _NC * _NS

_PER_W = BATCH // _NW
_CB = 8
_NCHUNK = _PER_W // _CB
_NBUF = 6
_LAG = 3


def _body(idx_hbm, table_hbm, out_hbm, idx_v, bufs, gsems, osems):
    wid = lax.axis_index("s") * _NC + lax.axis_index("c")
    base = wid * _PER_W

    pltpu.sync_copy(idx_hbm.at[pl.ds(base, _PER_W)], idx_v)

    def gather_one(j, i, b):
        return pltpu.make_async_copy(
            table_hbm.at[idx_v.at[j * _CB + i]], bufs.at[b, i], gsems.at[b])

    def gather_start(j, b):
        for i in range(_CB):
            gather_one(j, i, b).start()

    def gather_wait(j, b):
        for i in range(_CB):
            gather_one(j, i, b).wait()

    def put(j, b):
        return pltpu.make_async_copy(
            bufs.at[b], out_hbm.at[pl.ds(base + j * _CB, _CB)], osems.at[b])

    for t in range(_NBUF):
        gather_start(t, t % _NBUF)
        s = t - _LAG
        if s >= 0:
            gather_wait(s, s % _NBUF)
            put(s, s % _NBUF).start()

    def step(t, carry):
        b = t % _NBUF
        put(t - _NBUF, b).wait()
        gather_start(t, b)
        s = t - _LAG
        bs = s % _NBUF
        gather_wait(s, bs)
        put(s, bs).start()
        return carry

    lax.fori_loop(_NBUF, _NCHUNK, step, 0)

    for s in range(_NCHUNK - _LAG, _NCHUNK):
        gather_wait(s, s % _NBUF)
        put(s, s % _NBUF).start()
    for s in range(_NCHUNK - _NBUF, _NCHUNK):
        put(s, s % _NBUF).wait()


@jax.jit
def _lookup(idx, table):
    mesh = plsc.VectorSubcoreMesh(core_axis_name="c", subcore_axis_name="s")
    k = pl.kernel(
        _body,
        out_type=jax.ShapeDtypeStruct((BATCH, HIST, EMBED_DIM), jnp.float32),
        mesh=mesh,
        scratch_types=dict(
            idx_v=pltpu.VMEM((_PER_W, HIST), jnp.int32),
            bufs=pltpu.VMEM((_NBUF, _CB, HIST, EMBED_DIM), jnp.float32),
            gsems=pltpu.SemaphoreType.DMA((_NBUF,)),
            osems=pltpu.SemaphoreType.DMA((_NBUF,)),
        ),
        compiler_params=pltpu.CompilerParams(use_tc_tiling_on_sc=False),
    )
    return k(idx, table)


def kernel(id, W_center, W_context):
    return _lookup(id.astype(jnp.int32), W_center)
